# Initial kernel scaffold; baseline (speedup 1.0000x reference)
import jax
import jax.numpy as jnp
from jax.experimental import pallas as pl


def kernel(X, edge_index, edge_type, w_bases0, w_rel0, w_bases1, w_rel1):
    raise NotImplementedError("write your pallas kernel here")


# speedup vs baseline: 1.8703x; 1.8703x over previous
import functools

import jax
import jax.numpy as jnp
from jax import lax
from jax.experimental import pallas as pl
from jax.experimental.pallas import tpu as pltpu

N = 10000
E = 640000
NUM_REL = 4
NUM_BASES = 2
HIDDEN = 64
OUT = 64

_NBLK = 1000


def _prep_body(wb0_ref, wr0_ref, ind_ref, t0_ref, bias_ref):
    wr0 = wr0_ref[...]
    for r in range(NUM_REL):
        acc = jnp.zeros((_NBLK, HIDDEN), jnp.float32)
        for b in range(NUM_BASES):
            acc = acc + wr0[r, b] * wb0_ref[b]
        t0_ref[r] = acc


def _bias_body(wb0f_ref, wr0_ref, ind_ref, bias_ref):
    wr0 = wr0_ref[...]
    colsum = jnp.sum(wb0f_ref[...], axis=1)
    s = jnp.dot(wr0, colsum, precision=jax.lax.Precision.HIGHEST)
    bias_ref[...] = jnp.dot(ind_ref[...], s, precision=jax.lax.Precision.HIGHEST)


def _mm_body(h0a_ref, h0b_ref, bias_ref, wr1_ref, wb1_ref, t1_ref):
    h = jnp.maximum(h0a_ref[...] + h0b_ref[...] + bias_ref[...], 0.0)
    wr1 = wr1_ref[...]
    for r in range(NUM_REL):
        w = jnp.zeros((HIDDEN, OUT), jnp.float32)
        for b in range(NUM_BASES):
            w = w + wr1[r, b] * wb1_ref[b]
        t1_ref[r] = jnp.dot(h, w, preferred_element_type=jnp.float32,
                            precision=jax.lax.Precision.HIGHEST)


def _relu_mm(h0a, h0b, bias, w_rel1, w_bases1):
    grid = (N // _NBLK,)
    return pl.pallas_call(
        _mm_body,
        grid=grid,
        in_specs=[
            pl.BlockSpec((_NBLK, HIDDEN), lambda j: (j, 0)),
            pl.BlockSpec((_NBLK, HIDDEN), lambda j: (j, 0)),
            pl.BlockSpec((_NBLK, HIDDEN), lambda j: (j, 0)),
            pl.BlockSpec((NUM_REL, NUM_BASES), lambda j: (0, 0)),
            pl.BlockSpec((NUM_BASES, HIDDEN, OUT), lambda j: (0, 0, 0)),
        ],
        out_specs=pl.BlockSpec((NUM_REL, _NBLK, OUT), lambda j: (0, j, 0)),
        out_shape=jax.ShapeDtypeStruct((NUM_REL, N, OUT), jnp.float32),
    )(h0a, h0b, bias, w_rel1, w_bases1)


def _prep_t0(w_bases0, w_rel0):
    wb0_id = w_bases0[:, OUT:, :]
    grid = (N // _NBLK,)
    return pl.pallas_call(
        lambda wb_ref, wr_ref, t0_ref: _prep_body(wb_ref, wr_ref, None, t0_ref, None),
        grid=grid,
        in_specs=[
            pl.BlockSpec((NUM_BASES, _NBLK, HIDDEN), lambda j: (0, j, 0)),
            pl.BlockSpec((NUM_REL, NUM_BASES), lambda j: (0, 0)),
        ],
        out_specs=pl.BlockSpec((NUM_REL, _NBLK, HIDDEN), lambda j: (0, j, 0)),
        out_shape=jax.ShapeDtypeStruct((NUM_REL, N, HIDDEN), jnp.float32),
    )(wb0_id, w_rel0)


def _prep_bias(w_bases0, w_rel0, ind):
    wb0_f = w_bases0[:, :OUT, :]
    grid = (N // _NBLK,)
    return pl.pallas_call(
        _bias_body,
        grid=grid,
        in_specs=[
            pl.BlockSpec((NUM_BASES, OUT, HIDDEN), lambda j: (0, 0, 0)),
            pl.BlockSpec((NUM_REL, NUM_BASES), lambda j: (0, 0)),
            pl.BlockSpec((_NBLK, NUM_REL), lambda j: (j, 0)),
        ],
        out_specs=pl.BlockSpec((_NBLK, HIDDEN), lambda j: (j, 0)),
        out_shape=jax.ShapeDtypeStruct((N, HIDDEN), jnp.float32),
    )(wb0_f, w_rel0, ind)


def kernel(X, edge_index, edge_type, w_bases0, w_rel0, w_bases1, w_rel1):
    src, dst = edge_index[0], edge_index[1]
    t = edge_type

    deg = jnp.zeros((NUM_REL, N), jnp.float32).at[t, dst].add(1.0)
    recip = jnp.where(deg > 0, 1.0 / jnp.maximum(deg, 1.0), 0.0)
    coef = recip[t, dst]
    ind = (deg > 0).astype(jnp.float32).T

    t0 = _prep_t0(w_bases0, w_rel0)
    bias = _prep_bias(w_bases0, w_rel0, ind)

    msg0 = t0[t, src] * coef[:, None]
    h0 = jnp.zeros((N, HIDDEN), jnp.float32).at[dst].add(msg0)

    t1 = _relu_mm(h0, jnp.zeros_like(h0), bias, w_rel1, w_bases1)

    msg1 = t1[t, src] * coef[:, None]
    out = jnp.zeros((N, OUT), jnp.float32).at[dst].add(msg1)

    y = out[None]
    p = jnp.ones((1, N), jnp.float32)
    lam = jnp.ones((1, N), jnp.float32)
    return (y, p, lam)


# speedup vs baseline: 16.2118x; 8.6680x over previous
import functools

import jax
import jax.numpy as jnp
from jax import lax
from jax.experimental import pallas as pl
from jax.experimental.pallas import tpu as pltpu
from jax.experimental.pallas import tpu_sc as plsc

N = 10000
E = 640000
NUM_REL = 4
NUM_BASES = 2
HIDDEN = 64
OUT = 64

NW = 32
TPW = 20480
EP = NW * TPW
B = 1024
NCH = TPW // B
NG = B // 16
KD = B // 128

ACC0_ROWS = 10112
DEG_ROWS = 40960
DEG_STRIPE = DEG_ROWS // 16
A0_STRIPE = ACC0_ROWS // 16

_NBLK = 1000

_HI = jax.lax.Precision.HIGHEST
_mesh = plsc.VectorSubcoreMesh(core_axis_name="c", subcore_axis_name="s")
_SC_PARAMS = pltpu.CompilerParams(needs_layout_passes=False,
                                  use_tc_tiling_on_sc=False)


DEG_WORDS = 40016


def _deg_body(dst_hbm, t_hbm, outd, dstv, tv, degv):
    c = lax.axis_index("c")
    s = lax.axis_index("s")
    wid = c * 16 + s

    lanes = lax.iota(jnp.int32, 16)
    zero16 = jnp.zeros((16,), jnp.float32)
    ones16 = jnp.ones((16,), jnp.float32)

    def zdeg(g, _):
        degv[pl.ds(g * 16, 16)] = zero16
        return 0

    lax.fori_loop(0, DEG_WORDS // 16, zdeg, 0)

    def chunk(ci, _):
        base = wid * TPW + ci * B
        pltpu.sync_copy(dst_hbm.at[pl.ds(base, B)], dstv)
        pltpu.sync_copy(t_hbm.at[pl.ds(base, B)], tv)

        def grp(g, _):
            d16 = dstv[pl.ds(g * 16, 16)]
            t16 = tv[pl.ds(g * 16, 16)]
            gi = d16 * 4 + t16
            for j in range(16):
                plsc.addupdate_scatter(degv, [gi], ones16, mask=lanes == j)
            return 0

        lax.fori_loop(0, NG, grp, 0)
        return 0

    lax.fori_loop(0, NCH, chunk, 0)
    pltpu.sync_copy(degv, outd.at[pl.ds(wid * DEG_WORDS, DEG_WORDS)])


def _deg_counts(dstp, tp):
    f = pl.kernel(
        _deg_body,
        out_type=jax.ShapeDtypeStruct((NW * DEG_WORDS,), jnp.float32),
        mesh=_mesh,
        compiler_params=_SC_PARAMS,
        scratch_types=[
            pltpu.VMEM((B,), jnp.int32),
            pltpu.VMEM((B,), jnp.int32),
            pltpu.VMEM((DEG_WORDS,), jnp.float32),
        ],
    )
    return f(dstp, tp)


HALF = N // 2
ACC_ROWS = 5120
HW = 32
EPT = EP // 16
NCH2 = EPT // B


def _edge_body(src_hbm, dst_hbm, t_hbm, table_hbm, recip_hbm, outp,
               srcv, dstv, tv, gidx, sidx, coefv, rows, recipv, acc):
    c = lax.axis_index("c")
    s = lax.axis_index("s")
    lanes = lax.iota(jnp.int32, 16)
    lo = c * HALF

    pltpu.sync_copy(recip_hbm, recipv.at[pl.ds(0, 4 * N)])
    recipv[pl.ds(4 * N, 16)] = jnp.zeros((16,), jnp.float32)

    zero16 = jnp.zeros((16,), jnp.float32)

    def zrow(i, _):
        for k in range(HW // 16):
            rows[i, pl.ds(k * 16, 16)] = zero16
        return 0

    lax.fori_loop(0, B, zrow, 0)
    pltpu.sync_copy(rows.at[pl.ds(0, 320)], acc.at[pl.ds(s * 320, 320)])
    plsc.subcore_barrier()

    def chunk(ci, _):
        base = s * EPT + ci * B
        pltpu.sync_copy(src_hbm.at[pl.ds(base, B)], srcv)
        pltpu.sync_copy(dst_hbm.at[pl.ds(base, B)], dstv)
        pltpu.sync_copy(t_hbm.at[pl.ds(base, B)], tv)

        def grp(g, _):
            s16 = srcv[pl.ds(g * 16, 16)]
            d16 = dstv[pl.ds(g * 16, 16)]
            t16 = tv[pl.ds(g * 16, 16)]
            u16 = d16 - lo
            in_rng = (u16 >= 0) & (u16 < HALF)
            gidx[g // 8, pl.ds((g % 8) * 16, 16)] = t16 * N + s16
            sidx[g // 8, pl.ds((g % 8) * 16, 16)] = jnp.where(
                in_rng, u16, HALF + lanes)
            coefv[pl.ds(g * 16, 16)] = plsc.load_gather(recipv, [d16 * 4 + t16])
            return 0

        lax.fori_loop(0, NG, grp, 0)

        for j in range(KD):
            pltpu.sync_copy(table_hbm.at[gidx.at[j]],
                            rows.at[pl.ds(j * 128, 128)])

        def scale(g, _):
            for jj in range(16):
                e = g * 16 + jj
                cof = plsc.load_gather(coefv, [jnp.zeros((16,), jnp.int32) + e])
                for k in range(HW // 16):
                    rows[e, pl.ds(k * 16, 16)] = rows[e, pl.ds(k * 16, 16)] * cof
            return 0

        lax.fori_loop(0, NG, scale, 0)

        for j in range(KD):
            pltpu.sync_copy(rows.at[pl.ds(j * 128, 128)],
                            acc.at[sidx.at[j]], add=True)
        return 0

    lax.fori_loop(0, NCH2, chunk, 0)
    plsc.subcore_barrier()

    pltpu.sync_copy(acc.at[pl.ds(s * 320, 320)],
                    outp.at[c, pl.ds(s * 320, 320)])


def _edge_pass(srcp, dstp, tp, table, recip_flat):
    f = pl.kernel(
        _edge_body,
        out_type=jax.ShapeDtypeStruct((2, ACC_ROWS, HW), jnp.float32),
        mesh=_mesh,
        compiler_params=_SC_PARAMS,
        scratch_types=[
            pltpu.VMEM((B,), jnp.int32),
            pltpu.VMEM((B,), jnp.int32),
            pltpu.VMEM((B,), jnp.int32),
            pltpu.VMEM((KD, 128), jnp.int32),
            pltpu.VMEM((KD, 128), jnp.int32),
            pltpu.VMEM((B,), jnp.float32),
            pltpu.VMEM((B, HW), jnp.float32),
            pltpu.VMEM((4 * N + 16,), jnp.float32),
            pltpu.VMEM_SHARED((ACC_ROWS, HW), jnp.float32),
        ],
    )
    return f(srcp, dstp, tp, table, recip_flat)


def _prep_body(parts_ref, wb0f_ref, wb0id_ref, wr0_ref,
               recip_ref, bias_ref, t0_ref):
    d = jnp.sum(parts_ref[...], axis=0)
    recip_ref[...] = jnp.where(d > 0, 1.0 / jnp.maximum(d, 1.0), 0.0)
    ind = (d > 0).astype(jnp.float32)
    wr0 = wr0_ref[...]
    colsum = jnp.sum(wb0f_ref[...], axis=1)
    sm = jnp.dot(wr0, colsum, precision=_HI)
    bias_ref[...] = jnp.dot(ind, sm, precision=_HI)
    for r in range(NUM_REL):
        accv = wr0[r, 0] * wb0id_ref[0]
        for b in range(1, NUM_BASES):
            accv = accv + wr0[r, b] * wb0id_ref[b]
        t0_ref[r] = accv


def _prep(deg_parts, w_bases0, w_rel0):
    wb0f = w_bases0[:, :OUT, :]
    wb0id = w_bases0[:, OUT:, :]
    grid = (N // _NBLK,)
    return pl.pallas_call(
        _prep_body,
        grid=grid,
        in_specs=[
            pl.BlockSpec((NW, _NBLK, NUM_REL), lambda j: (0, j, 0)),
            pl.BlockSpec((NUM_BASES, OUT, HIDDEN), lambda j: (0, 0, 0)),
            pl.BlockSpec((NUM_BASES, _NBLK, HIDDEN), lambda j: (0, j, 0)),
            pl.BlockSpec((NUM_REL, NUM_BASES), lambda j: (0, 0)),
        ],
        out_specs=[
            pl.BlockSpec((_NBLK, NUM_REL), lambda j: (j, 0)),
            pl.BlockSpec((_NBLK, HIDDEN), lambda j: (j, 0)),
            pl.BlockSpec((NUM_REL, _NBLK, HIDDEN), lambda j: (0, j, 0)),
        ],
        out_shape=[
            jax.ShapeDtypeStruct((N, NUM_REL), jnp.float32),
            jax.ShapeDtypeStruct((N, HIDDEN), jnp.float32),
            jax.ShapeDtypeStruct((NUM_REL, N, HIDDEN), jnp.float32),
        ],
    )(deg_parts, wb0f, wb0id, w_rel0)


def _mid_body(h0_ref, bias_ref, wr1_ref, wb1_ref, t1_ref):
    h = jnp.maximum(h0_ref[...] + bias_ref[...], 0.0)
    wr1 = wr1_ref[...]
    for r in range(NUM_REL):
        w = wr1[r, 0] * wb1_ref[0]
        for b in range(1, NUM_BASES):
            w = w + wr1[r, b] * wb1_ref[b]
        t1_ref[r] = jnp.dot(h, w, preferred_element_type=jnp.float32,
                            precision=_HI)


def _mid(h0, bias, w_rel1, w_bases1):
    grid = (N // _NBLK,)
    return pl.pallas_call(
        _mid_body,
        grid=grid,
        in_specs=[
            pl.BlockSpec((_NBLK, HIDDEN), lambda j: (j, 0)),
            pl.BlockSpec((_NBLK, HIDDEN), lambda j: (j, 0)),
            pl.BlockSpec((NUM_REL, NUM_BASES), lambda j: (0, 0)),
            pl.BlockSpec((NUM_BASES, HIDDEN, OUT), lambda j: (0, 0, 0)),
        ],
        out_specs=pl.BlockSpec((NUM_REL, _NBLK, OUT), lambda j: (0, j, 0)),
        out_shape=jax.ShapeDtypeStruct((NUM_REL, N, OUT), jnp.float32),
    )(h0, bias, w_rel1, w_bases1)


def _fin_body(a_ref, b_ref, y_ref):
    y_ref[0] = a_ref[...] + b_ref[...]


def _fin(a, b):
    grid = (N // _NBLK,)
    return pl.pallas_call(
        _fin_body,
        grid=grid,
        in_specs=[
            pl.BlockSpec((_NBLK, OUT), lambda j: (j, 0)),
            pl.BlockSpec((_NBLK, OUT), lambda j: (j, 0)),
        ],
        out_specs=pl.BlockSpec((1, _NBLK, OUT), lambda j: (0, j, 0)),
        out_shape=jax.ShapeDtypeStruct((1, N, OUT), jnp.float32),
    )(a, b)


def kernel(X, edge_index, edge_type, w_bases0, w_rel0, w_bases1, w_rel1):
    src, dst = edge_index[0], edge_index[1]
    pad = EP - E
    srcp = jnp.concatenate([src, jnp.zeros((pad,), jnp.int32)])
    dstp = jnp.concatenate([dst, jnp.full((pad,), N, jnp.int32)])
    tp = jnp.concatenate([edge_type, jnp.zeros((pad,), jnp.int32)])

    degp = _deg_counts(dstp, tp)
    deg_parts = degp.reshape(NW, DEG_WORDS)[:, : 4 * N].reshape(NW, N, NUM_REL)

    recip, bias, t0 = _prep(deg_parts, w_bases0, w_rel0)
    recip_flat = recip.reshape(4 * N)
    t0_flat = t0.reshape(NUM_REL * N, HIDDEN)

    h0pa = _edge_pass(srcp, dstp, tp, t0_flat[:, :HW], recip_flat)
    h0pb = _edge_pass(srcp, dstp, tp, t0_flat[:, HW:], recip_flat)
    h0 = jnp.concatenate(
        [h0pa[:, :HALF, :].reshape(N, HW), h0pb[:, :HALF, :].reshape(N, HW)],
        axis=1)
    t1 = _mid(h0, bias, w_rel1, w_bases1)
    t1_flat = t1.reshape(NUM_REL * N, OUT)

    outpa = _edge_pass(srcp, dstp, tp, t1_flat[:, :HW], recip_flat)
    outpb = _edge_pass(srcp, dstp, tp, t1_flat[:, HW:], recip_flat)
    y = jnp.concatenate(
        [outpa[:, :HALF, :].reshape(N, HW), outpb[:, :HALF, :].reshape(N, HW)],
        axis=1).reshape(1, N, OUT)

    p = jnp.ones((1, N), jnp.float32)
    lam = jnp.ones((1, N), jnp.float32)
    return (y, p, lam)


# speedup vs baseline: 21.0399x; 1.2978x over previous
import functools

import jax
import jax.numpy as jnp
from jax import lax
from jax.experimental import pallas as pl
from jax.experimental.pallas import tpu as pltpu
from jax.experimental.pallas import tpu_sc as plsc

N = 10000
E = 640000
NUM_REL = 4
NUM_BASES = 2
HIDDEN = 64
OUT = 64

NW = 32
TPW = 20480
EP = NW * TPW
B = 1024
NCH = TPW // B
NG = B // 16
KD = B // 128

ACC0_ROWS = 10112
DEG_ROWS = 40960
DEG_STRIPE = DEG_ROWS // 16
A0_STRIPE = ACC0_ROWS // 16

_NBLK = 1000

_HI = jax.lax.Precision.HIGHEST
_mesh = plsc.VectorSubcoreMesh(core_axis_name="c", subcore_axis_name="s")
_SC_PARAMS = pltpu.CompilerParams(needs_layout_passes=False,
                                  use_tc_tiling_on_sc=False)


DEG_WORDS = 40016


def _deg_body(dst_hbm, t_hbm, outd, dstv, tv, degv):
    c = lax.axis_index("c")
    s = lax.axis_index("s")
    wid = c * 16 + s

    lanes = lax.iota(jnp.int32, 16)
    zero16 = jnp.zeros((16,), jnp.float32)
    ones16 = jnp.ones((16,), jnp.float32)

    def zdeg(g, _):
        degv[pl.ds(g * 16, 16)] = zero16
        return 0

    lax.fori_loop(0, DEG_WORDS // 16, zdeg, 0)

    def chunk(ci, _):
        base = wid * TPW + ci * B
        pltpu.sync_copy(dst_hbm.at[pl.ds(base, B)], dstv)
        pltpu.sync_copy(t_hbm.at[pl.ds(base, B)], tv)

        def grp(g, _):
            d16 = dstv[pl.ds(g * 16, 16)]
            t16 = tv[pl.ds(g * 16, 16)]
            gi = d16 * 4 + t16
            for j in range(16):
                plsc.addupdate_scatter(degv, [gi], ones16, mask=lanes == j)
            return 0

        lax.fori_loop(0, NG, grp, 0)
        return 0

    lax.fori_loop(0, NCH, chunk, 0)
    pltpu.sync_copy(degv, outd.at[pl.ds(wid * DEG_WORDS, DEG_WORDS)])


def _deg_counts(dstp, tp):
    f = pl.kernel(
        _deg_body,
        out_type=jax.ShapeDtypeStruct((NW * DEG_WORDS,), jnp.float32),
        mesh=_mesh,
        compiler_params=_SC_PARAMS,
        scratch_types=[
            pltpu.VMEM((B,), jnp.int32),
            pltpu.VMEM((B,), jnp.int32),
            pltpu.VMEM((DEG_WORDS,), jnp.float32),
        ],
    )
    return f(dstp, tp)


HALF = N // 2
ACC_ROWS = 5120
HW = 32
EPT = EP // 16
NCH2 = EPT // B


def _edge_body(src_hbm, dst_hbm, t_hbm, table_hbm, recip_hbm, outp,
               srcv, dstv, tv, gidx, sidx, coefv, rows, recipv, acc,
               sem_l, sem_g, sem_s):
    c = lax.axis_index("c")
    s = lax.axis_index("s")
    lanes = lax.iota(jnp.int32, 16)
    lo = c * HALF

    pltpu.sync_copy(recip_hbm, recipv.at[pl.ds(0, 4 * N)])
    recipv[pl.ds(4 * N, 16)] = jnp.zeros((16,), jnp.float32)

    zero16 = jnp.zeros((16,), jnp.float32)

    def zrow(i, _):
        for k in range(HW // 16):
            rows[i, pl.ds(k * 16, 16)] = zero16
        return 0

    lax.fori_loop(0, B, zrow, 0)
    pltpu.sync_copy(rows.at[pl.ds(0, 320)], acc.at[pl.ds(s * 320, 320)])
    plsc.subcore_barrier()

    def chunk(ci, _):
        base = s * EPT + ci * B
        dls = [pltpu.async_copy(src_hbm.at[pl.ds(base, B)], srcv, sem_l),
               pltpu.async_copy(dst_hbm.at[pl.ds(base, B)], dstv, sem_l),
               pltpu.async_copy(t_hbm.at[pl.ds(base, B)], tv, sem_l)]
        for d in dls:
            d.wait()

        def grp(g, _):
            s16 = srcv[pl.ds(g * 16, 16)]
            d16 = dstv[pl.ds(g * 16, 16)]
            t16 = tv[pl.ds(g * 16, 16)]
            u16 = d16 - lo
            in_rng = (u16 >= 0) & (u16 < HALF)
            gidx[g // 8, pl.ds((g % 8) * 16, 16)] = t16 * N + s16
            sidx[g // 8, pl.ds((g % 8) * 16, 16)] = jnp.where(
                in_rng, u16, HALF + lanes)
            coefv[pl.ds(g * 16, 16)] = plsc.load_gather(recipv, [d16 * 4 + t16])
            return 0

        lax.fori_loop(0, NG, grp, 0)

        dgs = [pltpu.async_copy(table_hbm.at[gidx.at[j]],
                                rows.at[pl.ds(j * 128, 128)], sem_g)
               for j in range(KD)]
        for d in dgs:
            d.wait()

        def scale(g, _):
            for jj in range(16):
                e = g * 16 + jj
                cof = plsc.load_gather(coefv, [jnp.zeros((16,), jnp.int32) + e])
                for k in range(HW // 16):
                    rows[e, pl.ds(k * 16, 16)] = rows[e, pl.ds(k * 16, 16)] * cof
            return 0

        lax.fori_loop(0, NG, scale, 0)

        dss = [pltpu.async_copy(rows.at[pl.ds(j * 128, 128)],
                                acc.at[sidx.at[j]], sem_s, add=True)
               for j in range(KD)]
        for d in dss:
            d.wait()
        return 0

    lax.fori_loop(0, NCH2, chunk, 0)
    plsc.subcore_barrier()

    pltpu.sync_copy(acc.at[pl.ds(s * 320, 320)],
                    outp.at[c, pl.ds(s * 320, 320)])


def _edge_pass(srcp, dstp, tp, table, recip_flat):
    f = pl.kernel(
        _edge_body,
        out_type=jax.ShapeDtypeStruct((2, ACC_ROWS, HW), jnp.float32),
        mesh=_mesh,
        compiler_params=_SC_PARAMS,
        scratch_types=[
            pltpu.VMEM((B,), jnp.int32),
            pltpu.VMEM((B,), jnp.int32),
            pltpu.VMEM((B,), jnp.int32),
            pltpu.VMEM((KD, 128), jnp.int32),
            pltpu.VMEM((KD, 128), jnp.int32),
            pltpu.VMEM((B,), jnp.float32),
            pltpu.VMEM((B, HW), jnp.float32),
            pltpu.VMEM((4 * N + 16,), jnp.float32),
            pltpu.VMEM_SHARED((ACC_ROWS, HW), jnp.float32),
            pltpu.SemaphoreType.DMA,
            pltpu.SemaphoreType.DMA,
            pltpu.SemaphoreType.DMA,
        ],
    )
    return f(srcp, dstp, tp, table, recip_flat)


def _prep_body(parts_ref, wb0f_ref, wb0id_ref, wr0_ref,
               recip_ref, bias_ref, t0_ref):
    d = jnp.sum(parts_ref[...], axis=0)
    recip_ref[...] = jnp.where(d > 0, 1.0 / jnp.maximum(d, 1.0), 0.0)
    ind = (d > 0).astype(jnp.float32)
    wr0 = wr0_ref[...]
    colsum = jnp.sum(wb0f_ref[...], axis=1)
    sm = jnp.dot(wr0, colsum, precision=_HI)
    bias_ref[...] = jnp.dot(ind, sm, precision=_HI)
    for r in range(NUM_REL):
        accv = wr0[r, 0] * wb0id_ref[0]
        for b in range(1, NUM_BASES):
            accv = accv + wr0[r, b] * wb0id_ref[b]
        t0_ref[r] = accv


def _prep(deg_parts, w_bases0, w_rel0):
    wb0f = w_bases0[:, :OUT, :]
    wb0id = w_bases0[:, OUT:, :]
    grid = (N // _NBLK,)
    return pl.pallas_call(
        _prep_body,
        grid=grid,
        in_specs=[
            pl.BlockSpec((NW, _NBLK, NUM_REL), lambda j: (0, j, 0)),
            pl.BlockSpec((NUM_BASES, OUT, HIDDEN), lambda j: (0, 0, 0)),
            pl.BlockSpec((NUM_BASES, _NBLK, HIDDEN), lambda j: (0, j, 0)),
            pl.BlockSpec((NUM_REL, NUM_BASES), lambda j: (0, 0)),
        ],
        out_specs=[
            pl.BlockSpec((_NBLK, NUM_REL), lambda j: (j, 0)),
            pl.BlockSpec((_NBLK, HIDDEN), lambda j: (j, 0)),
            pl.BlockSpec((NUM_REL, _NBLK, HIDDEN), lambda j: (0, j, 0)),
        ],
        out_shape=[
            jax.ShapeDtypeStruct((N, NUM_REL), jnp.float32),
            jax.ShapeDtypeStruct((N, HIDDEN), jnp.float32),
            jax.ShapeDtypeStruct((NUM_REL, N, HIDDEN), jnp.float32),
        ],
    )(deg_parts, wb0f, wb0id, w_rel0)


def _mid_body(h0_ref, bias_ref, wr1_ref, wb1_ref, t1_ref):
    h = jnp.maximum(h0_ref[...] + bias_ref[...], 0.0)
    wr1 = wr1_ref[...]
    for r in range(NUM_REL):
        w = wr1[r, 0] * wb1_ref[0]
        for b in range(1, NUM_BASES):
            w = w + wr1[r, b] * wb1_ref[b]
        t1_ref[r] = jnp.dot(h, w, preferred_element_type=jnp.float32,
                            precision=_HI)


def _mid(h0, bias, w_rel1, w_bases1):
    grid = (N // _NBLK,)
    return pl.pallas_call(
        _mid_body,
        grid=grid,
        in_specs=[
            pl.BlockSpec((_NBLK, HIDDEN), lambda j: (j, 0)),
            pl.BlockSpec((_NBLK, HIDDEN), lambda j: (j, 0)),
            pl.BlockSpec((NUM_REL, NUM_BASES), lambda j: (0, 0)),
            pl.BlockSpec((NUM_BASES, HIDDEN, OUT), lambda j: (0, 0, 0)),
        ],
        out_specs=pl.BlockSpec((NUM_REL, _NBLK, OUT), lambda j: (0, j, 0)),
        out_shape=jax.ShapeDtypeStruct((NUM_REL, N, OUT), jnp.float32),
    )(h0, bias, w_rel1, w_bases1)


def _fin_body(a_ref, b_ref, y_ref):
    y_ref[0] = a_ref[...] + b_ref[...]


def _fin(a, b):
    grid = (N // _NBLK,)
    return pl.pallas_call(
        _fin_body,
        grid=grid,
        in_specs=[
            pl.BlockSpec((_NBLK, OUT), lambda j: (j, 0)),
            pl.BlockSpec((_NBLK, OUT), lambda j: (j, 0)),
        ],
        out_specs=pl.BlockSpec((1, _NBLK, OUT), lambda j: (0, j, 0)),
        out_shape=jax.ShapeDtypeStruct((1, N, OUT), jnp.float32),
    )(a, b)


def kernel(X, edge_index, edge_type, w_bases0, w_rel0, w_bases1, w_rel1):
    src, dst = edge_index[0], edge_index[1]
    pad = EP - E
    srcp = jnp.concatenate([src, jnp.zeros((pad,), jnp.int32)])
    dstp = jnp.concatenate([dst, jnp.full((pad,), N, jnp.int32)])
    tp = jnp.concatenate([edge_type, jnp.zeros((pad,), jnp.int32)])

    degp = _deg_counts(dstp, tp)
    deg_parts = degp.reshape(NW, DEG_WORDS)[:, : 4 * N].reshape(NW, N, NUM_REL)

    recip, bias, t0 = _prep(deg_parts, w_bases0, w_rel0)
    recip_flat = recip.reshape(4 * N)
    t0_flat = t0.reshape(NUM_REL * N, HIDDEN)

    h0pa = _edge_pass(srcp, dstp, tp, t0_flat[:, :HW], recip_flat)
    h0pb = _edge_pass(srcp, dstp, tp, t0_flat[:, HW:], recip_flat)
    h0 = jnp.concatenate(
        [h0pa[:, :HALF, :].reshape(N, HW), h0pb[:, :HALF, :].reshape(N, HW)],
        axis=1)
    t1 = _mid(h0, bias, w_rel1, w_bases1)
    t1_flat = t1.reshape(NUM_REL * N, OUT)

    outpa = _edge_pass(srcp, dstp, tp, t1_flat[:, :HW], recip_flat)
    outpb = _edge_pass(srcp, dstp, tp, t1_flat[:, HW:], recip_flat)
    y = jnp.concatenate(
        [outpa[:, :HALF, :].reshape(N, HW), outpb[:, :HALF, :].reshape(N, HW)],
        axis=1).reshape(1, N, OUT)

    p = jnp.ones((1, N), jnp.float32)
    lam = jnp.ones((1, N), jnp.float32)
    return (y, p, lam)
